# Initial kernel scaffold; baseline (speedup 1.0000x reference)
import jax
import jax.numpy as jnp
from jax.experimental import pallas as pl


def kernel(x, edge_index, W_self, b_self, W_agg, b_agg, gamma, beta):
    raise NotImplementedError("write your pallas kernel here")


# speedup vs baseline: 5.3253x; 5.3253x over previous
import functools

import jax
import jax.numpy as jnp
from jax import lax
from jax.experimental import pallas as pl
from jax.experimental.pallas import tpu as pltpu
from jax.experimental.pallas import tpu_sc as plsc

N = 10000
E = 320000
D = 128

NC = 2
NS = 16
NW = NC * NS

EDGES_PER_WORKER = E // NW
CHUNK = 128
FULL_CHUNKS = EDGES_PER_WORKER // CHUNK
TAIL = EDGES_PER_WORKER - FULL_CHUNKS * CHUNK

NPAD = 10240
ROWS_PER_SUBCORE = NPAD // NS
CNTW = 128

_MESH = plsc.VectorSubcoreMesh(
    core_axis_name="c", subcore_axis_name="s", num_cores=NC,
    num_subcores=NS)


def _sc_feature_sums(x, row, col, zeros_feat):
  @functools.partial(
      pl.kernel,
      out_type=jax.ShapeDtypeStruct((NC, NPAD, D), jnp.float32),
      mesh=_MESH,
      scratch_types=[
          pltpu.VMEM((1, CHUNK), jnp.int32),
          pltpu.VMEM((1, CHUNK), jnp.int32),
          pltpu.VMEM((1, TAIL), jnp.int32),
          pltpu.VMEM((1, TAIL), jnp.int32),
          pltpu.VMEM((CHUNK, D), jnp.float32),
          pltpu.VMEM_SHARED((NPAD, D), jnp.float32),
          pltpu.SemaphoreType.DMA,
      ],
  )
  def sc_kernel(x_hbm, row_hbm, col_hbm, zf_hbm, out_sum,
                colv, rowv, colt, rowt, gathv, acc, sem):
    cid = lax.axis_index("c")
    sid = lax.axis_index("s")
    wid = cid * NS + sid

    pltpu.sync_copy(zf_hbm, gathv)
    zbase = sid * ROWS_PER_SUBCORE

    @pl.loop(0, ROWS_PER_SUBCORE // CHUNK)
    def _(j):
      pltpu.sync_copy(gathv, acc.at[pl.ds(zbase + j * CHUNK, CHUNK)])

    plsc.subcore_barrier()

    ebase = wid * EDGES_PER_WORKER

    @pl.loop(0, FULL_CHUNKS)
    def _(i):
      off = ebase + i * CHUNK
      pltpu.sync_copy(col_hbm.at[pl.ds(off, CHUNK)], colv.at[0])
      pltpu.sync_copy(row_hbm.at[pl.ds(off, CHUNK)], rowv.at[0])
      pltpu.async_copy(x_hbm.at[colv.at[0]], gathv, sem).wait()
      pltpu.sync_copy(gathv, acc.at[rowv.at[0]], add=True)

    toff = ebase + FULL_CHUNKS * CHUNK
    pltpu.sync_copy(col_hbm.at[pl.ds(toff, TAIL)], colt.at[0])
    pltpu.sync_copy(row_hbm.at[pl.ds(toff, TAIL)], rowt.at[0])
    pltpu.async_copy(x_hbm.at[colt.at[0]], gathv.at[pl.ds(0, TAIL)],
                     sem).wait()
    pltpu.sync_copy(gathv.at[pl.ds(0, TAIL)], acc.at[rowt.at[0]], add=True)

    plsc.subcore_barrier()

    @pl.loop(0, ROWS_PER_SUBCORE // CHUNK)
    def _(j):
      b = zbase + j * CHUNK
      pltpu.sync_copy(acc.at[pl.ds(b, CHUNK)], gathv)
      pltpu.sync_copy(gathv, out_sum.at[cid, pl.ds(b, CHUNK)])

  return sc_kernel(x, row, col, zeros_feat)


def _sc_degree_counts(row, zeros_cnt, ones_cnt):
  @functools.partial(
      pl.kernel,
      out_type=jax.ShapeDtypeStruct((NC, NPAD, CNTW), jnp.float32),
      mesh=_MESH,
      scratch_types=[
          pltpu.VMEM((1, CHUNK), jnp.int32),
          pltpu.VMEM((1, TAIL), jnp.int32),
          pltpu.VMEM((CHUNK, CNTW), jnp.float32),
          pltpu.VMEM_SHARED((NPAD, CNTW), jnp.float32),
      ],
  )
  def sc_kernel(row_hbm, zc_hbm, on_hbm, out_cnt,
                rowv, rowt, onesv, cnt):
    cid = lax.axis_index("c")
    sid = lax.axis_index("s")
    wid = cid * NS + sid

    pltpu.sync_copy(zc_hbm, onesv)
    zbase = sid * ROWS_PER_SUBCORE

    @pl.loop(0, ROWS_PER_SUBCORE // CHUNK)
    def _(j):
      pltpu.sync_copy(onesv, cnt.at[pl.ds(zbase + j * CHUNK, CHUNK)])

    pltpu.sync_copy(on_hbm, onesv)
    plsc.subcore_barrier()

    ebase = wid * EDGES_PER_WORKER

    @pl.loop(0, FULL_CHUNKS)
    def _(i):
      off = ebase + i * CHUNK
      pltpu.sync_copy(row_hbm.at[pl.ds(off, CHUNK)], rowv.at[0])
      pltpu.sync_copy(onesv, cnt.at[rowv.at[0]], add=True)

    toff = ebase + FULL_CHUNKS * CHUNK
    pltpu.sync_copy(row_hbm.at[pl.ds(toff, TAIL)], rowt.at[0])
    pltpu.sync_copy(onesv.at[pl.ds(0, TAIL)], cnt.at[rowt.at[0]], add=True)

    plsc.subcore_barrier()

    @pl.loop(0, ROWS_PER_SUBCORE // CHUNK)
    def _(j):
      b = zbase + j * CHUNK
      pltpu.sync_copy(cnt.at[pl.ds(b, CHUNK)], onesv)
      pltpu.sync_copy(onesv, out_cnt.at[cid, pl.ds(b, CHUNK)])

  return sc_kernel(row, zeros_cnt, ones_cnt)


BLK = 1000


def _tc_body(x_ref, ps_ref, pc_ref, ws_ref, wa_ref, bs_ref, ba_ref,
             g_ref, b_ref, out_ref):
  s = ps_ref[0] + ps_ref[1]
  c = pc_ref[0, :, 0:1] + pc_ref[1, :, 0:1]
  mean = s / (c + 1e-8)
  h = jnp.dot(x_ref[...], ws_ref[...], preferred_element_type=jnp.float32)
  h = h + jnp.dot(mean, wa_ref[...], preferred_element_type=jnp.float32)
  h = h + bs_ref[...] + ba_ref[...]
  h = jnp.maximum(h, 0.0)
  mu = jnp.mean(h, axis=1, keepdims=True)
  var = jnp.mean((h - mu) ** 2, axis=1, keepdims=True)
  out_ref[...] = (h - mu) * lax.rsqrt(var + 1e-5) * g_ref[...] + b_ref[...]


def _tc_finish(x, psum, pcnt, w_self_t, w_agg_t, b_self, b_agg, gamma, beta):
  grid = (N // BLK,)
  full128 = pl.BlockSpec((1, D), lambda i: (0, 0))
  return pl.pallas_call(
      _tc_body,
      grid=grid,
      in_specs=[
          pl.BlockSpec((BLK, D), lambda i: (i, 0)),
          pl.BlockSpec((NC, BLK, D), lambda i: (0, i, 0)),
          pl.BlockSpec((NC, BLK, CNTW), lambda i: (0, i, 0)),
          pl.BlockSpec((D, D), lambda i: (0, 0)),
          pl.BlockSpec((D, D), lambda i: (0, 0)),
          full128, full128, full128, full128,
      ],
      out_specs=pl.BlockSpec((BLK, D), lambda i: (i, 0)),
      out_shape=jax.ShapeDtypeStruct((N, D), jnp.float32),
  )(x, psum, pcnt, w_self_t, w_agg_t,
    b_self.reshape(1, D), b_agg.reshape(1, D),
    gamma.reshape(1, D), beta.reshape(1, D))


@jax.jit
def kernel(x, edge_index, W_self, b_self, W_agg, b_agg, gamma, beta):
  row = edge_index[0]
  col = edge_index[1]
  zeros_feat = jnp.zeros((CHUNK, D), jnp.float32)
  zeros_cnt = jnp.zeros((CHUNK, CNTW), jnp.float32)
  ones_cnt = jnp.ones((CHUNK, CNTW), jnp.float32)
  psum = _sc_feature_sums(x, row, col, zeros_feat)
  pcnt = _sc_degree_counts(row, zeros_cnt, ones_cnt)
  return _tc_finish(x, psum, pcnt, W_self.T, W_agg.T,
                    b_self, b_agg, gamma, beta)


# speedup vs baseline: 8.9009x; 1.6714x over previous
import functools

import jax
import jax.numpy as jnp
from jax import lax
from jax.experimental import pallas as pl
from jax.experimental.pallas import tpu as pltpu
from jax.experimental.pallas import tpu_sc as plsc

N = 10000
E = 320000
D = 128

NC = 2
NS = 16
NW = NC * NS

EDGES_PER_WORKER = E // NW
CHUNK = 80
NCHUNK = EDGES_PER_WORKER // CHUNK
assert NCHUNK * CHUNK == EDGES_PER_WORKER

NPAD = 10240
ROWS_PER_SUBCORE = NPAD // NS
ZCH = ROWS_PER_SUBCORE // CHUNK

_MESH = plsc.VectorSubcoreMesh(
    core_axis_name="c", subcore_axis_name="s", num_cores=NC,
    num_subcores=NS)


def _sc_feature_sums(x, row, col, zeros_feat):
  @functools.partial(
      pl.kernel,
      out_type=jax.ShapeDtypeStruct((NC, NPAD, D), jnp.float32),
      mesh=_MESH,
      scratch_types=[
          [pltpu.VMEM((1, CHUNK), jnp.int32) for _ in range(4)],
          [pltpu.VMEM((1, CHUNK), jnp.int32) for _ in range(4)],
          [pltpu.VMEM((CHUNK, D), jnp.float32) for _ in range(2)],
          pltpu.VMEM_SHARED((NPAD, D), jnp.float32),
          [pltpu.SemaphoreType.DMA for _ in range(4)],
          [pltpu.SemaphoreType.DMA for _ in range(2)],
          [pltpu.SemaphoreType.DMA for _ in range(2)],
      ],
  )
  def sc_kernel(x_hbm, row_hbm, col_hbm, zf_hbm, out_sum,
                colv, rowv, gv, acc, sem_i, sem_g, sem_s):
    cid = lax.axis_index("c")
    sid = lax.axis_index("s")
    wid = cid * NS + sid
    ebase = wid * EDGES_PER_WORKER
    zbase = sid * ROWS_PER_SUBCORE

    pltpu.sync_copy(zf_hbm, gv[0])

    @pl.loop(0, ZCH)
    def _(j):
      pltpu.sync_copy(gv[0], acc.at[pl.ds(zbase + j * CHUNK, CHUNK)])

    plsc.subcore_barrier()

    def issue_idx(c, s):
      off = ebase + c * CHUNK
      pltpu.async_copy(col_hbm.at[pl.ds(off, CHUNK)], colv[s].at[0],
                       sem_i[s])
      pltpu.async_copy(row_hbm.at[pl.ds(off, CHUNK)], rowv[s].at[0],
                       sem_i[s])

    def wait_idx(s):
      pltpu.make_async_copy(col_hbm.at[pl.ds(0, CHUNK)], colv[s].at[0],
                            sem_i[s]).wait()
      pltpu.make_async_copy(row_hbm.at[pl.ds(0, CHUNK)], rowv[s].at[0],
                            sem_i[s]).wait()

    def issue_gather(s, g):
      pltpu.async_copy(x_hbm.at[colv[s].at[0]], gv[g], sem_g[g])

    def wait_gather(s, g):
      pltpu.make_async_copy(x_hbm.at[colv[s].at[0]], gv[g],
                            sem_g[g]).wait()

    def issue_scatter(g, s):
      pltpu.async_copy(gv[g], acc.at[rowv[s].at[0]], sem_s[g], add=True)

    def wait_scatter(g, s):
      pltpu.make_async_copy(gv[g], acc.at[rowv[s].at[0]],
                            sem_s[g]).wait()

    def visit(c, i_cur, i_nxt, i_pre, g_cur, g_nxt,
              first=False, do_gather=True, do_idx=True):
      if not first:
        wait_scatter(g_nxt, i_pre)
      if do_gather:
        wait_idx(i_nxt)
        issue_gather(i_nxt, g_nxt)
      wait_gather(i_cur, g_cur)
      issue_scatter(g_cur, i_cur)
      if do_idx:
        issue_idx(c + 3, i_pre)

    issue_idx(0, 0)
    issue_idx(1, 1)
    issue_idx(2, 2)
    wait_idx(0)
    issue_gather(0, 0)
    visit(0, 0, 1, 3, 0, 1, first=True)

    @pl.loop(0, 30)
    def _(j):
      c = 1 + 4 * j
      visit(c + 0, 1, 2, 0, 1, 0)
      visit(c + 1, 2, 3, 1, 0, 1)
      visit(c + 2, 3, 0, 2, 1, 0)
      visit(c + 3, 0, 1, 3, 0, 1)

    visit(121, 1, 2, 0, 1, 0)
    visit(122, 2, 3, 1, 0, 1, do_idx=False)
    visit(123, 3, 0, 2, 1, 0, do_idx=False)
    visit(124, 0, 1, 3, 0, 1, do_gather=False, do_idx=False)
    wait_scatter(0, 0)

    plsc.subcore_barrier()

    @pl.loop(0, ZCH)
    def _(j):
      b = zbase + j * CHUNK
      pltpu.sync_copy(acc.at[pl.ds(b, CHUNK)], gv[0])
      pltpu.sync_copy(gv[0], out_sum.at[cid, pl.ds(b, CHUNK)])

  return sc_kernel(x, row, col, zeros_feat)


def _sc_degree_counts(row, zeros_feat, ones_feat):
  @functools.partial(
      pl.kernel,
      out_type=jax.ShapeDtypeStruct((NC, NPAD, D), jnp.float32),
      mesh=_MESH,
      scratch_types=[
          [pltpu.VMEM((1, CHUNK), jnp.int32) for _ in range(4)],
          pltpu.VMEM((CHUNK, D), jnp.float32),
          pltpu.VMEM_SHARED((NPAD, D), jnp.float32),
          [pltpu.SemaphoreType.DMA for _ in range(4)],
          [pltpu.SemaphoreType.DMA for _ in range(2)],
      ],
  )
  def sc_kernel(row_hbm, zf_hbm, on_hbm, out_cnt,
                rowv, onesv, cnt, sem_i, sem_s):
    cid = lax.axis_index("c")
    sid = lax.axis_index("s")
    wid = cid * NS + sid
    ebase = wid * EDGES_PER_WORKER
    zbase = sid * ROWS_PER_SUBCORE

    pltpu.sync_copy(zf_hbm, onesv)

    @pl.loop(0, ZCH)
    def _(j):
      pltpu.sync_copy(onesv, cnt.at[pl.ds(zbase + j * CHUNK, CHUNK)])

    pltpu.sync_copy(on_hbm, onesv)
    plsc.subcore_barrier()

    def issue_idx(c, s):
      off = ebase + c * CHUNK
      pltpu.async_copy(row_hbm.at[pl.ds(off, CHUNK)], rowv[s].at[0],
                       sem_i[s])

    def wait_idx(s):
      pltpu.make_async_copy(row_hbm.at[pl.ds(0, CHUNK)], rowv[s].at[0],
                            sem_i[s]).wait()

    def issue_scatter(s, g):
      pltpu.async_copy(onesv, cnt.at[rowv[s].at[0]], sem_s[g], add=True)

    def wait_scatter(s, g):
      pltpu.make_async_copy(onesv, cnt.at[rowv[s].at[0]],
                            sem_s[g]).wait()

    def visit(c, i_cur, i_pre, g_cur, g_nxt, first=False, do_idx=True):
      if not first:
        wait_scatter(i_pre, g_nxt)
      wait_idx(i_cur)
      issue_scatter(i_cur, g_cur)
      if do_idx:
        issue_idx(c + 3, i_pre)

    issue_idx(0, 0)
    issue_idx(1, 1)
    issue_idx(2, 2)
    visit(0, 0, 3, 0, 1, first=True)

    @pl.loop(0, 30)
    def _(j):
      c = 1 + 4 * j
      visit(c + 0, 1, 0, 1, 0)
      visit(c + 1, 2, 1, 0, 1)
      visit(c + 2, 3, 2, 1, 0)
      visit(c + 3, 0, 3, 0, 1)

    visit(121, 1, 0, 1, 0)
    visit(122, 2, 1, 0, 1, do_idx=False)
    visit(123, 3, 2, 1, 0, do_idx=False)
    visit(124, 0, 3, 0, 1, do_idx=False)
    wait_scatter(0, 0)

    plsc.subcore_barrier()

    @pl.loop(0, ZCH)
    def _(j):
      b = zbase + j * CHUNK
      pltpu.sync_copy(cnt.at[pl.ds(b, CHUNK)], onesv)
      pltpu.sync_copy(onesv, out_cnt.at[cid, pl.ds(b, CHUNK)])

  return sc_kernel(row, zeros_feat, ones_feat)


BLK = 1000


def _tc_body(x_ref, ps_ref, pc_ref, ws_ref, wa_ref, bs_ref, ba_ref,
             g_ref, b_ref, out_ref):
  s = ps_ref[0] + ps_ref[1]
  c = pc_ref[0, :, 0:1] + pc_ref[1, :, 0:1]
  mean = s / (c + 1e-8)
  h = jnp.dot(x_ref[...], ws_ref[...], preferred_element_type=jnp.float32)
  h = h + jnp.dot(mean, wa_ref[...], preferred_element_type=jnp.float32)
  h = h + bs_ref[...] + ba_ref[...]
  h = jnp.maximum(h, 0.0)
  mu = jnp.mean(h, axis=1, keepdims=True)
  var = jnp.mean((h - mu) ** 2, axis=1, keepdims=True)
  out_ref[...] = (h - mu) * lax.rsqrt(var + 1e-5) * g_ref[...] + b_ref[...]


def _tc_finish(x, psum, pcnt, w_self_t, w_agg_t, b_self, b_agg, gamma, beta):
  grid = (N // BLK,)
  full128 = pl.BlockSpec((1, D), lambda i: (0, 0))
  return pl.pallas_call(
      _tc_body,
      grid=grid,
      in_specs=[
          pl.BlockSpec((BLK, D), lambda i: (i, 0)),
          pl.BlockSpec((NC, BLK, D), lambda i: (0, i, 0)),
          pl.BlockSpec((NC, BLK, D), lambda i: (0, i, 0)),
          pl.BlockSpec((D, D), lambda i: (0, 0)),
          pl.BlockSpec((D, D), lambda i: (0, 0)),
          full128, full128, full128, full128,
      ],
      out_specs=pl.BlockSpec((BLK, D), lambda i: (i, 0)),
      out_shape=jax.ShapeDtypeStruct((N, D), jnp.float32),
  )(x, psum, pcnt, w_self_t, w_agg_t,
    b_self.reshape(1, D), b_agg.reshape(1, D),
    gamma.reshape(1, D), beta.reshape(1, D))


@jax.jit
def kernel(x, edge_index, W_self, b_self, W_agg, b_agg, gamma, beta):
  row = edge_index[0]
  col = edge_index[1]
  zeros_feat = jnp.zeros((CHUNK, D), jnp.float32)
  ones_feat = jnp.ones((CHUNK, D), jnp.float32)
  psum = _sc_feature_sums(x, row, col, zeros_feat)
  pcnt = _sc_degree_counts(row, zeros_feat, ones_feat)
  return _tc_finish(x, psum, pcnt, W_self.T, W_agg.T,
                    b_self, b_agg, gamma, beta)


# speedup vs baseline: 9.6182x; 1.0806x over previous
import functools

import jax
import jax.numpy as jnp
from jax import lax
from jax.experimental import pallas as pl
from jax.experimental.pallas import tpu as pltpu
from jax.experimental.pallas import tpu_sc as plsc

N = 10000
E = 320000
D = 128

NC = 2
NS = 16
NW = NC * NS

EDGES_PER_WORKER = E // NW
CHUNK = 80
NCHUNK = EDGES_PER_WORKER // CHUNK
assert NCHUNK * CHUNK == EDGES_PER_WORKER

NPAD = 10240
ROWS_PER_SUBCORE = NPAD // NS
ZCH = ROWS_PER_SUBCORE // CHUNK

_MESH = plsc.VectorSubcoreMesh(
    core_axis_name="c", subcore_axis_name="s", num_cores=NC,
    num_subcores=NS)


def _sc_feature_sums(x, row, col, zeros_feat):
  @functools.partial(
      pl.kernel,
      out_type=jax.ShapeDtypeStruct((NC, NPAD, D), jnp.float32),
      mesh=_MESH,
      scratch_types=[
          [pltpu.VMEM((1, CHUNK), jnp.int32) for _ in range(4)],
          [pltpu.VMEM((1, CHUNK), jnp.int32) for _ in range(4)],
          [pltpu.VMEM((CHUNK, D), jnp.float32) for _ in range(3)],
          pltpu.VMEM_SHARED((NPAD, D), jnp.float32),
          [pltpu.SemaphoreType.DMA for _ in range(4)],
          [pltpu.SemaphoreType.DMA for _ in range(3)],
          [pltpu.SemaphoreType.DMA for _ in range(3)],
      ],
  )
  def sc_kernel(x_hbm, row_hbm, col_hbm, zf_hbm, out_sum,
                colv, rowv, gv, acc, sem_i, sem_g, sem_s):
    cid = lax.axis_index("c")
    sid = lax.axis_index("s")
    wid = cid * NS + sid
    ebase = wid * EDGES_PER_WORKER
    zbase = sid * ROWS_PER_SUBCORE

    pltpu.sync_copy(zf_hbm, gv[0])

    @pl.loop(0, ZCH)
    def _(j):
      pltpu.sync_copy(gv[0], acc.at[pl.ds(zbase + j * CHUNK, CHUNK)])

    plsc.subcore_barrier()

    def issue_idx(c, s):
      off = ebase + c * CHUNK
      pltpu.async_copy(col_hbm.at[pl.ds(off, CHUNK)], colv[s].at[0],
                       sem_i[s])
      pltpu.async_copy(row_hbm.at[pl.ds(off, CHUNK)], rowv[s].at[0],
                       sem_i[s])

    def wait_idx(s):
      pltpu.make_async_copy(col_hbm.at[pl.ds(0, CHUNK)], colv[s].at[0],
                            sem_i[s]).wait()
      pltpu.make_async_copy(row_hbm.at[pl.ds(0, CHUNK)], rowv[s].at[0],
                            sem_i[s]).wait()

    def issue_gather(s, g):
      pltpu.async_copy(x_hbm.at[colv[s].at[0]], gv[g], sem_g[g])

    def wait_gather(s, g):
      pltpu.make_async_copy(x_hbm.at[colv[s].at[0]], gv[g],
                            sem_g[g]).wait()

    def issue_scatter(g, s):
      pltpu.async_copy(gv[g], acc.at[rowv[s].at[0]], sem_s[g], add=True)

    def wait_scatter(g, s):
      pltpu.make_async_copy(gv[g], acc.at[rowv[s].at[0]],
                            sem_s[g]).wait()

    def visit(c, cm, first=False, do_gather=True, do_idx=True):
      i_cur, i_nxt, i_pre = cm % 4, (cm + 1) % 4, (cm + 3) % 4
      g_cur, g_nxt, g_prv = cm % 3, (cm + 1) % 3, (cm + 2) % 3
      if do_gather:
        wait_idx(i_nxt)
        issue_gather(i_nxt, g_nxt)
      wait_gather(i_cur, g_cur)
      issue_scatter(g_cur, i_cur)
      if not first:
        wait_scatter(g_prv, i_pre)
      if do_idx:
        issue_idx(c + 3, i_pre)

    issue_idx(0, 0)
    issue_idx(1, 1)
    issue_idx(2, 2)
    wait_idx(0)
    issue_gather(0, 0)
    visit(0, 0, first=True)

    @pl.loop(0, 10)
    def _(j):
      c = 1 + 12 * j
      for u in range(12):
        visit(c + u, 1 + u)

    visit(121, 121)
    visit(122, 122, do_idx=False)
    visit(123, 123, do_idx=False)
    visit(124, 124, do_gather=False, do_idx=False)
    wait_scatter(124 % 3, 124 % 4)

    plsc.subcore_barrier()

    @pl.loop(0, ZCH)
    def _(j):
      b = zbase + j * CHUNK
      pltpu.sync_copy(acc.at[pl.ds(b, CHUNK)], gv[0])
      pltpu.sync_copy(gv[0], out_sum.at[cid, pl.ds(b, CHUNK)])

  return sc_kernel(x, row, col, zeros_feat)


def _sc_degree_counts(row, zeros_feat, ones_feat):
  @functools.partial(
      pl.kernel,
      out_type=jax.ShapeDtypeStruct((NC, NPAD, D), jnp.float32),
      mesh=_MESH,
      scratch_types=[
          [pltpu.VMEM((1, CHUNK), jnp.int32) for _ in range(4)],
          pltpu.VMEM((CHUNK, D), jnp.float32),
          pltpu.VMEM_SHARED((NPAD, D), jnp.float32),
          [pltpu.SemaphoreType.DMA for _ in range(4)],
          [pltpu.SemaphoreType.DMA for _ in range(2)],
      ],
  )
  def sc_kernel(row_hbm, zf_hbm, on_hbm, out_cnt,
                rowv, onesv, cnt, sem_i, sem_s):
    cid = lax.axis_index("c")
    sid = lax.axis_index("s")
    wid = cid * NS + sid
    ebase = wid * EDGES_PER_WORKER
    zbase = sid * ROWS_PER_SUBCORE

    pltpu.sync_copy(zf_hbm, onesv)

    @pl.loop(0, ZCH)
    def _(j):
      pltpu.sync_copy(onesv, cnt.at[pl.ds(zbase + j * CHUNK, CHUNK)])

    pltpu.sync_copy(on_hbm, onesv)
    plsc.subcore_barrier()

    def issue_idx(c, s):
      off = ebase + c * CHUNK
      pltpu.async_copy(row_hbm.at[pl.ds(off, CHUNK)], rowv[s].at[0],
                       sem_i[s])

    def wait_idx(s):
      pltpu.make_async_copy(row_hbm.at[pl.ds(0, CHUNK)], rowv[s].at[0],
                            sem_i[s]).wait()

    def issue_scatter(s, g):
      pltpu.async_copy(onesv, cnt.at[rowv[s].at[0]], sem_s[g], add=True)

    def wait_scatter(s, g):
      pltpu.make_async_copy(onesv, cnt.at[rowv[s].at[0]],
                            sem_s[g]).wait()

    def visit(c, i_cur, i_pre, g_cur, g_nxt, first=False, do_idx=True):
      if not first:
        wait_scatter(i_pre, g_nxt)
      wait_idx(i_cur)
      issue_scatter(i_cur, g_cur)
      if do_idx:
        issue_idx(c + 3, i_pre)

    issue_idx(0, 0)
    issue_idx(1, 1)
    issue_idx(2, 2)
    visit(0, 0, 3, 0, 1, first=True)

    @pl.loop(0, 30)
    def _(j):
      c = 1 + 4 * j
      visit(c + 0, 1, 0, 1, 0)
      visit(c + 1, 2, 1, 0, 1)
      visit(c + 2, 3, 2, 1, 0)
      visit(c + 3, 0, 3, 0, 1)

    visit(121, 1, 0, 1, 0)
    visit(122, 2, 1, 0, 1, do_idx=False)
    visit(123, 3, 2, 1, 0, do_idx=False)
    visit(124, 0, 3, 0, 1, do_idx=False)
    wait_scatter(0, 0)

    plsc.subcore_barrier()

    @pl.loop(0, ZCH)
    def _(j):
      b = zbase + j * CHUNK
      pltpu.sync_copy(cnt.at[pl.ds(b, CHUNK)], onesv)
      pltpu.sync_copy(onesv, out_cnt.at[cid, pl.ds(b, CHUNK)])

  return sc_kernel(row, zeros_feat, ones_feat)


BLK = 1000


def _tc_body(x_ref, ps_ref, pc_ref, ws_ref, wa_ref, bs_ref, ba_ref,
             g_ref, b_ref, out_ref):
  s = ps_ref[0] + ps_ref[1]
  c = pc_ref[0, :, 0:1] + pc_ref[1, :, 0:1]
  mean = s / (c + 1e-8)
  h = jnp.dot(x_ref[...], ws_ref[...], preferred_element_type=jnp.float32)
  h = h + jnp.dot(mean, wa_ref[...], preferred_element_type=jnp.float32)
  h = h + bs_ref[...] + ba_ref[...]
  h = jnp.maximum(h, 0.0)
  mu = jnp.mean(h, axis=1, keepdims=True)
  var = jnp.mean((h - mu) ** 2, axis=1, keepdims=True)
  out_ref[...] = (h - mu) * lax.rsqrt(var + 1e-5) * g_ref[...] + b_ref[...]


def _tc_finish(x, psum, pcnt, w_self_t, w_agg_t, b_self, b_agg, gamma, beta):
  grid = (N // BLK,)
  full128 = pl.BlockSpec((1, D), lambda i: (0, 0))
  return pl.pallas_call(
      _tc_body,
      grid=grid,
      in_specs=[
          pl.BlockSpec((BLK, D), lambda i: (i, 0)),
          pl.BlockSpec((NC, BLK, D), lambda i: (0, i, 0)),
          pl.BlockSpec((NC, BLK, D), lambda i: (0, i, 0)),
          pl.BlockSpec((D, D), lambda i: (0, 0)),
          pl.BlockSpec((D, D), lambda i: (0, 0)),
          full128, full128, full128, full128,
      ],
      out_specs=pl.BlockSpec((BLK, D), lambda i: (i, 0)),
      out_shape=jax.ShapeDtypeStruct((N, D), jnp.float32),
  )(x, psum, pcnt, w_self_t, w_agg_t,
    b_self.reshape(1, D), b_agg.reshape(1, D),
    gamma.reshape(1, D), beta.reshape(1, D))


@jax.jit
def kernel(x, edge_index, W_self, b_self, W_agg, b_agg, gamma, beta):
  row = edge_index[0]
  col = edge_index[1]
  zeros_feat = jnp.zeros((CHUNK, D), jnp.float32)
  ones_feat = jnp.ones((CHUNK, D), jnp.float32)
  psum = _sc_feature_sums(x, row, col, zeros_feat)
  pcnt = _sc_degree_counts(row, zeros_feat, ones_feat)
  return _tc_finish(x, psum, pcnt, W_self.T, W_agg.T,
                    b_self, b_agg, gamma, beta)


# speedup vs baseline: 9.6187x; 1.0001x over previous
import functools

import jax
import jax.numpy as jnp
from jax import lax
from jax.experimental import pallas as pl
from jax.experimental.pallas import tpu as pltpu
from jax.experimental.pallas import tpu_sc as plsc

N = 10000
E = 320000
D = 128

NC = 2
NS = 16
NW = NC * NS

EDGES_PER_WORKER = E // NW
CHUNK = 80
NCHUNK = EDGES_PER_WORKER // CHUNK
assert NCHUNK * CHUNK == EDGES_PER_WORKER

NPAD = 10240
ROWS_PER_SUBCORE = NPAD // NS
ZCH = ROWS_PER_SUBCORE // CHUNK

_MESH = plsc.VectorSubcoreMesh(
    core_axis_name="c", subcore_axis_name="s", num_cores=NC,
    num_subcores=NS)


def _sc_feature_sums(x, row, col, zeros_feat):
  @functools.partial(
      pl.kernel,
      out_type=jax.ShapeDtypeStruct((NC, NPAD, D), jnp.float32),
      mesh=_MESH,
      scratch_types=[
          [pltpu.VMEM((1, CHUNK), jnp.int32) for _ in range(4)],
          [pltpu.VMEM((1, CHUNK), jnp.int32) for _ in range(4)],
          [pltpu.VMEM((CHUNK, D), jnp.float32) for _ in range(3)],
          pltpu.VMEM_SHARED((NPAD, D), jnp.float32),
          [pltpu.SemaphoreType.DMA for _ in range(4)],
          [pltpu.SemaphoreType.DMA for _ in range(3)],
          [pltpu.SemaphoreType.DMA for _ in range(3)],
      ],
  )
  def sc_kernel(x_hbm, row_hbm, col_hbm, zf_hbm, out_sum,
                colv, rowv, gv, acc, sem_i, sem_g, sem_s):
    cid = lax.axis_index("c")
    sid = lax.axis_index("s")
    wid = cid * NS + sid
    ebase = wid * EDGES_PER_WORKER
    zbase = sid * ROWS_PER_SUBCORE

    pltpu.sync_copy(zf_hbm, gv[0])

    @pl.loop(0, ZCH)
    def _(j):
      pltpu.sync_copy(gv[0], acc.at[pl.ds(zbase + j * CHUNK, CHUNK)])

    plsc.subcore_barrier()

    def issue_idx(c, s):
      off = ebase + c * CHUNK
      pltpu.async_copy(col_hbm.at[pl.ds(off, CHUNK)], colv[s].at[0],
                       sem_i[s])
      pltpu.async_copy(row_hbm.at[pl.ds(off, CHUNK)], rowv[s].at[0],
                       sem_i[s])

    def wait_idx(s):
      pltpu.make_async_copy(col_hbm.at[pl.ds(0, CHUNK)], colv[s].at[0],
                            sem_i[s]).wait()
      pltpu.make_async_copy(row_hbm.at[pl.ds(0, CHUNK)], rowv[s].at[0],
                            sem_i[s]).wait()

    def issue_gather(s, g):
      pltpu.async_copy(x_hbm.at[colv[s].at[0]], gv[g], sem_g[g])

    def wait_gather(s, g):
      pltpu.make_async_copy(x_hbm.at[colv[s].at[0]], gv[g],
                            sem_g[g]).wait()

    def issue_scatter(g, s):
      pltpu.async_copy(gv[g], acc.at[rowv[s].at[0]], sem_s[g], add=True)

    def wait_scatter(g, s):
      pltpu.make_async_copy(gv[g], acc.at[rowv[s].at[0]],
                            sem_s[g]).wait()

    def visit(c, cm, first=False, do_gather=True, do_idx=True):
      i_cur, i_nxt, i_pre = cm % 4, (cm + 1) % 4, (cm + 3) % 4
      g_cur, g_nxt, g_prv = cm % 3, (cm + 1) % 3, (cm + 2) % 3
      if do_gather:
        wait_idx(i_nxt)
        issue_gather(i_nxt, g_nxt)
      wait_gather(i_cur, g_cur)
      issue_scatter(g_cur, i_cur)
      if not first:
        wait_scatter(g_prv, i_pre)
      if do_idx:
        issue_idx(c + 3, i_pre)

    issue_idx(0, 0)
    issue_idx(1, 1)
    issue_idx(2, 2)
    wait_idx(0)
    issue_gather(0, 0)
    visit(0, 0, first=True)

    @pl.loop(0, 10)
    def _(j):
      c = 1 + 12 * j
      for u in range(12):
        visit(c + u, 1 + u)

    visit(121, 121)
    visit(122, 122, do_idx=False)
    visit(123, 123, do_idx=False)
    visit(124, 124, do_gather=False, do_idx=False)
    wait_scatter(124 % 3, 124 % 4)

    plsc.subcore_barrier()

    @pl.loop(0, ZCH)
    def _(j):
      b = zbase + j * CHUNK
      pltpu.sync_copy(acc.at[pl.ds(b, CHUNK)], gv[0])
      pltpu.sync_copy(gv[0], out_sum.at[cid, pl.ds(b, CHUNK)])

  return sc_kernel(x, row, col, zeros_feat)


def _sc_degree_counts(row, zeros_feat, ones_feat):
  @functools.partial(
      pl.kernel,
      out_type=jax.ShapeDtypeStruct((NC, NPAD, D), jnp.float32),
      mesh=_MESH,
      scratch_types=[
          [pltpu.VMEM((1, CHUNK), jnp.int32) for _ in range(4)],
          pltpu.VMEM((CHUNK, D), jnp.float32),
          pltpu.VMEM_SHARED((NPAD, D), jnp.float32),
          [pltpu.SemaphoreType.DMA for _ in range(4)],
          [pltpu.SemaphoreType.DMA for _ in range(3)],
      ],
  )
  def sc_kernel(row_hbm, zf_hbm, on_hbm, out_cnt,
                rowv, onesv, cnt, sem_i, sem_s):
    cid = lax.axis_index("c")
    sid = lax.axis_index("s")
    wid = cid * NS + sid
    ebase = wid * EDGES_PER_WORKER
    zbase = sid * ROWS_PER_SUBCORE

    pltpu.sync_copy(zf_hbm, onesv)

    @pl.loop(0, ZCH)
    def _(j):
      pltpu.sync_copy(onesv, cnt.at[pl.ds(zbase + j * CHUNK, CHUNK)])

    pltpu.sync_copy(on_hbm, onesv)
    plsc.subcore_barrier()

    def issue_idx(c, s):
      off = ebase + c * CHUNK
      pltpu.async_copy(row_hbm.at[pl.ds(off, CHUNK)], rowv[s].at[0],
                       sem_i[s])

    def wait_idx(s):
      pltpu.make_async_copy(row_hbm.at[pl.ds(0, CHUNK)], rowv[s].at[0],
                            sem_i[s]).wait()

    def issue_scatter(s, g):
      pltpu.async_copy(onesv, cnt.at[rowv[s].at[0]], sem_s[g], add=True)

    def wait_scatter(s, g):
      pltpu.make_async_copy(onesv, cnt.at[rowv[s].at[0]],
                            sem_s[g]).wait()

    def visit(c, cm, first=False, do_idx=True):
      i_cur, i_pre = cm % 4, (cm + 3) % 4
      g_cur, g_prv = cm % 3, (cm + 2) % 3
      wait_idx(i_cur)
      issue_scatter(i_cur, g_cur)
      if not first:
        wait_scatter(i_pre, g_prv)
      if do_idx:
        issue_idx(c + 3, i_pre)

    issue_idx(0, 0)
    issue_idx(1, 1)
    issue_idx(2, 2)
    visit(0, 0, first=True)

    @pl.loop(0, 10)
    def _(j):
      c = 1 + 12 * j
      for u in range(12):
        visit(c + u, 1 + u)

    visit(121, 121)
    visit(122, 122, do_idx=False)
    visit(123, 123, do_idx=False)
    visit(124, 124, do_idx=False)
    wait_scatter(124 % 4, 124 % 3)

    plsc.subcore_barrier()

    @pl.loop(0, ZCH)
    def _(j):
      b = zbase + j * CHUNK
      pltpu.sync_copy(cnt.at[pl.ds(b, CHUNK)], onesv)
      pltpu.sync_copy(onesv, out_cnt.at[cid, pl.ds(b, CHUNK)])

  return sc_kernel(row, zeros_feat, ones_feat)


BLK = 1000


def _tc_self_body(x_ref, ws_ref, bs_ref, out_ref):
  out_ref[...] = (
      jnp.dot(x_ref[...], ws_ref[...], preferred_element_type=jnp.float32)
      + bs_ref[...])


def _tc_self(x, w_self_t, b_self):
  grid = (N // BLK,)
  return pl.pallas_call(
      _tc_self_body,
      grid=grid,
      in_specs=[
          pl.BlockSpec((BLK, D), lambda i: (i, 0)),
          pl.BlockSpec((D, D), lambda i: (0, 0)),
          pl.BlockSpec((1, D), lambda i: (0, 0)),
      ],
      out_specs=pl.BlockSpec((BLK, D), lambda i: (i, 0)),
      out_shape=jax.ShapeDtypeStruct((N, D), jnp.float32),
  )(x, w_self_t, b_self.reshape(1, D))


def _tc_body(sf_ref, ps_ref, pc_ref, wa_ref, ba_ref, g_ref, b_ref, out_ref):
  s = ps_ref[0] + ps_ref[1]
  c = pc_ref[0, :, 0:1] + pc_ref[1, :, 0:1]
  mean = s / (c + 1e-8)
  h = sf_ref[...]
  h = h + jnp.dot(mean, wa_ref[...], preferred_element_type=jnp.float32)
  h = h + ba_ref[...]
  h = jnp.maximum(h, 0.0)
  mu = jnp.mean(h, axis=1, keepdims=True)
  var = jnp.mean((h - mu) ** 2, axis=1, keepdims=True)
  out_ref[...] = (h - mu) * lax.rsqrt(var + 1e-5) * g_ref[...] + b_ref[...]


def _tc_finish(self_out, psum, pcnt, w_agg_t, b_agg, gamma, beta):
  grid = (N // BLK,)
  full128 = pl.BlockSpec((1, D), lambda i: (0, 0))
  return pl.pallas_call(
      _tc_body,
      grid=grid,
      in_specs=[
          pl.BlockSpec((BLK, D), lambda i: (i, 0)),
          pl.BlockSpec((NC, BLK, D), lambda i: (0, i, 0)),
          pl.BlockSpec((NC, BLK, D), lambda i: (0, i, 0)),
          pl.BlockSpec((D, D), lambda i: (0, 0)),
          full128, full128, full128,
      ],
      out_specs=pl.BlockSpec((BLK, D), lambda i: (i, 0)),
      out_shape=jax.ShapeDtypeStruct((N, D), jnp.float32),
  )(self_out, psum, pcnt, w_agg_t,
    b_agg.reshape(1, D), gamma.reshape(1, D), beta.reshape(1, D))


@jax.jit
def kernel(x, edge_index, W_self, b_self, W_agg, b_agg, gamma, beta):
  row = edge_index[0]
  col = edge_index[1]
  zeros_feat = jnp.zeros((CHUNK, D), jnp.float32)
  ones_feat = jnp.ones((CHUNK, D), jnp.float32)
  self_out = _tc_self(x, W_self.T, b_self)
  psum = _sc_feature_sums(x, row, col, zeros_feat)
  pcnt = _sc_degree_counts(row, zeros_feat, ones_feat)
  return _tc_finish(self_out, psum, pcnt, W_agg.T, b_agg, gamma, beta)


# speedup vs baseline: 9.7761x; 1.0164x over previous
import functools

import jax
import jax.numpy as jnp
from jax import lax
from jax.experimental import pallas as pl
from jax.experimental.pallas import tpu as pltpu
from jax.experimental.pallas import tpu_sc as plsc

N = 10000
E = 320000
D = 128

NC = 2
NS = 16
NW = NC * NS

EDGES_PER_WORKER = E // NW
CHUNK = 80
NCHUNK = EDGES_PER_WORKER // CHUNK
assert NCHUNK * CHUNK == EDGES_PER_WORKER

NPAD = 10240
ROWS_PER_SUBCORE = NPAD // NS
ZCH = ROWS_PER_SUBCORE // CHUNK

_MESH = plsc.VectorSubcoreMesh(
    core_axis_name="c", subcore_axis_name="s", num_cores=NC,
    num_subcores=NS)


def _sc_feature_sums(x, row, col, zeros_feat):
  @functools.partial(
      pl.kernel,
      out_type=jax.ShapeDtypeStruct((NC, NPAD, D), jnp.float32),
      mesh=_MESH,
      scratch_types=[
          [pltpu.VMEM((1, CHUNK), jnp.int32) for _ in range(4)],
          [pltpu.VMEM((1, CHUNK), jnp.int32) for _ in range(4)],
          [pltpu.VMEM((CHUNK, D), jnp.float32) for _ in range(3)],
          pltpu.VMEM_SHARED((NPAD, D), jnp.float32),
          [pltpu.SemaphoreType.DMA for _ in range(4)],
          [pltpu.SemaphoreType.DMA for _ in range(3)],
          [pltpu.SemaphoreType.DMA for _ in range(3)],
      ],
  )
  def sc_kernel(x_hbm, row_hbm, col_hbm, zf_hbm, out_sum,
                colv, rowv, gv, acc, sem_i, sem_g, sem_s):
    cid = lax.axis_index("c")
    sid = lax.axis_index("s")
    wid = cid * NS + sid
    ebase = wid * EDGES_PER_WORKER
    zbase = sid * ROWS_PER_SUBCORE

    pltpu.sync_copy(zf_hbm, gv[0])

    @pl.loop(0, ZCH)
    def _(j):
      pltpu.async_copy(gv[0], acc.at[pl.ds(zbase + j * CHUNK, CHUNK)],
                       sem_g[0])

    @pl.loop(0, ZCH)
    def _(j):
      pltpu.make_async_copy(gv[0], acc.at[pl.ds(zbase, CHUNK)],
                            sem_g[0]).wait()

    plsc.subcore_barrier()

    def issue_idx(c, s):
      off = ebase + c * CHUNK
      pltpu.async_copy(col_hbm.at[pl.ds(off, CHUNK)], colv[s].at[0],
                       sem_i[s])
      pltpu.async_copy(row_hbm.at[pl.ds(off, CHUNK)], rowv[s].at[0],
                       sem_i[s])

    def wait_idx(s):
      pltpu.make_async_copy(col_hbm.at[pl.ds(0, CHUNK)], colv[s].at[0],
                            sem_i[s]).wait()
      pltpu.make_async_copy(row_hbm.at[pl.ds(0, CHUNK)], rowv[s].at[0],
                            sem_i[s]).wait()

    def issue_gather(s, g):
      pltpu.async_copy(x_hbm.at[colv[s].at[0]], gv[g], sem_g[g])

    def wait_gather(s, g):
      pltpu.make_async_copy(x_hbm.at[colv[s].at[0]], gv[g],
                            sem_g[g]).wait()

    def issue_scatter(g, s):
      pltpu.async_copy(gv[g], acc.at[rowv[s].at[0]], sem_s[g], add=True)

    def wait_scatter(g, s):
      pltpu.make_async_copy(gv[g], acc.at[rowv[s].at[0]],
                            sem_s[g]).wait()

    def visit(c, cm, first=False, do_gather=True, do_idx=True):
      i_cur, i_nxt, i_pre = cm % 4, (cm + 1) % 4, (cm + 3) % 4
      g_cur, g_nxt, g_prv = cm % 3, (cm + 1) % 3, (cm + 2) % 3
      if do_gather:
        wait_idx(i_nxt)
        issue_gather(i_nxt, g_nxt)
      wait_gather(i_cur, g_cur)
      issue_scatter(g_cur, i_cur)
      if not first:
        wait_scatter(g_prv, i_pre)
      if do_idx:
        issue_idx(c + 3, i_pre)

    issue_idx(0, 0)
    issue_idx(1, 1)
    issue_idx(2, 2)
    wait_idx(0)
    issue_gather(0, 0)
    visit(0, 0, first=True)

    @pl.loop(0, 10)
    def _(j):
      c = 1 + 12 * j
      for u in range(12):
        visit(c + u, 1 + u)

    visit(121, 121)
    visit(122, 122, do_idx=False)
    visit(123, 123, do_idx=False)
    visit(124, 124, do_gather=False, do_idx=False)
    wait_scatter(124 % 3, 124 % 4)

    plsc.subcore_barrier()

    for j in range(ZCH):
      g = j % 3
      b = zbase + j * CHUNK
      if j >= 3:
        pltpu.make_async_copy(gv[g], out_sum.at[cid, pl.ds(zbase, CHUNK)],
                              sem_s[g]).wait()
      pltpu.sync_copy(acc.at[pl.ds(b, CHUNK)], gv[g])
      pltpu.async_copy(gv[g], out_sum.at[cid, pl.ds(b, CHUNK)], sem_s[g])
    for g in range(3):
      pltpu.make_async_copy(gv[g], out_sum.at[cid, pl.ds(zbase, CHUNK)],
                            sem_s[g]).wait()

  return sc_kernel(x, row, col, zeros_feat)


def _sc_degree_counts(row, zeros_feat, ones_feat):
  @functools.partial(
      pl.kernel,
      out_type=jax.ShapeDtypeStruct((NC, NPAD, D), jnp.float32),
      mesh=_MESH,
      scratch_types=[
          [pltpu.VMEM((1, CHUNK), jnp.int32) for _ in range(4)],
          pltpu.VMEM((CHUNK, D), jnp.float32),
          pltpu.VMEM_SHARED((NPAD, D), jnp.float32),
          [pltpu.SemaphoreType.DMA for _ in range(4)],
          [pltpu.SemaphoreType.DMA for _ in range(3)],
      ],
  )
  def sc_kernel(row_hbm, zf_hbm, on_hbm, out_cnt,
                rowv, onesv, cnt, sem_i, sem_s):
    cid = lax.axis_index("c")
    sid = lax.axis_index("s")
    wid = cid * NS + sid
    ebase = wid * EDGES_PER_WORKER
    zbase = sid * ROWS_PER_SUBCORE

    pltpu.sync_copy(zf_hbm, onesv)

    @pl.loop(0, ZCH)
    def _(j):
      pltpu.async_copy(onesv, cnt.at[pl.ds(zbase + j * CHUNK, CHUNK)],
                       sem_s[0])

    @pl.loop(0, ZCH)
    def _(j):
      pltpu.make_async_copy(onesv, cnt.at[pl.ds(zbase, CHUNK)],
                            sem_s[0]).wait()

    pltpu.sync_copy(on_hbm, onesv)
    plsc.subcore_barrier()

    def issue_idx(c, s):
      off = ebase + c * CHUNK
      pltpu.async_copy(row_hbm.at[pl.ds(off, CHUNK)], rowv[s].at[0],
                       sem_i[s])

    def wait_idx(s):
      pltpu.make_async_copy(row_hbm.at[pl.ds(0, CHUNK)], rowv[s].at[0],
                            sem_i[s]).wait()

    def issue_scatter(s, g):
      pltpu.async_copy(onesv, cnt.at[rowv[s].at[0]], sem_s[g], add=True)

    def wait_scatter(s, g):
      pltpu.make_async_copy(onesv, cnt.at[rowv[s].at[0]],
                            sem_s[g]).wait()

    def visit(c, cm, first=False, do_idx=True):
      i_cur, i_pre = cm % 4, (cm + 3) % 4
      g_cur, g_prv = cm % 3, (cm + 2) % 3
      wait_idx(i_cur)
      issue_scatter(i_cur, g_cur)
      if not first:
        wait_scatter(i_pre, g_prv)
      if do_idx:
        issue_idx(c + 3, i_pre)

    issue_idx(0, 0)
    issue_idx(1, 1)
    issue_idx(2, 2)
    visit(0, 0, first=True)

    @pl.loop(0, 10)
    def _(j):
      c = 1 + 12 * j
      for u in range(12):
        visit(c + u, 1 + u)

    visit(121, 121)
    visit(122, 122, do_idx=False)
    visit(123, 123, do_idx=False)
    visit(124, 124, do_idx=False)
    wait_scatter(124 % 4, 124 % 3)

    plsc.subcore_barrier()

    @pl.loop(0, ZCH)
    def _(j):
      b = zbase + j * CHUNK
      pltpu.sync_copy(cnt.at[pl.ds(b, CHUNK)], onesv)
      pltpu.sync_copy(onesv, out_cnt.at[cid, pl.ds(b, CHUNK)])

  return sc_kernel(row, zeros_feat, ones_feat)


BLK = 1000


def _tc_self_body(x_ref, ws_ref, bs_ref, out_ref):
  out_ref[...] = (
      jnp.dot(x_ref[...], ws_ref[...], preferred_element_type=jnp.float32)
      + bs_ref[...])


def _tc_self(x, w_self_t, b_self):
  grid = (N // BLK,)
  return pl.pallas_call(
      _tc_self_body,
      grid=grid,
      in_specs=[
          pl.BlockSpec((BLK, D), lambda i: (i, 0)),
          pl.BlockSpec((D, D), lambda i: (0, 0)),
          pl.BlockSpec((1, D), lambda i: (0, 0)),
      ],
      out_specs=pl.BlockSpec((BLK, D), lambda i: (i, 0)),
      out_shape=jax.ShapeDtypeStruct((N, D), jnp.float32),
  )(x, w_self_t, b_self.reshape(1, D))


def _tc_body(sf_ref, ps_ref, pc_ref, wa_ref, ba_ref, g_ref, b_ref, out_ref):
  s = ps_ref[0] + ps_ref[1]
  c = pc_ref[0, :, 0:1] + pc_ref[1, :, 0:1]
  mean = s / (c + 1e-8)
  h = sf_ref[...]
  h = h + jnp.dot(mean, wa_ref[...], preferred_element_type=jnp.float32)
  h = h + ba_ref[...]
  h = jnp.maximum(h, 0.0)
  mu = jnp.mean(h, axis=1, keepdims=True)
  var = jnp.mean((h - mu) ** 2, axis=1, keepdims=True)
  out_ref[...] = (h - mu) * lax.rsqrt(var + 1e-5) * g_ref[...] + b_ref[...]


def _tc_finish(self_out, psum, pcnt, w_agg_t, b_agg, gamma, beta):
  grid = (N // BLK,)
  full128 = pl.BlockSpec((1, D), lambda i: (0, 0))
  return pl.pallas_call(
      _tc_body,
      grid=grid,
      in_specs=[
          pl.BlockSpec((BLK, D), lambda i: (i, 0)),
          pl.BlockSpec((NC, BLK, D), lambda i: (0, i, 0)),
          pl.BlockSpec((NC, BLK, D), lambda i: (0, i, 0)),
          pl.BlockSpec((D, D), lambda i: (0, 0)),
          full128, full128, full128,
      ],
      out_specs=pl.BlockSpec((BLK, D), lambda i: (i, 0)),
      out_shape=jax.ShapeDtypeStruct((N, D), jnp.float32),
  )(self_out, psum, pcnt, w_agg_t,
    b_agg.reshape(1, D), gamma.reshape(1, D), beta.reshape(1, D))


@jax.jit
def kernel(x, edge_index, W_self, b_self, W_agg, b_agg, gamma, beta):
  row = edge_index[0]
  col = edge_index[1]
  zeros_feat = jnp.zeros((CHUNK, D), jnp.float32)
  ones_feat = jnp.ones((CHUNK, D), jnp.float32)
  self_out = _tc_self(x, W_self.T, b_self)
  psum = _sc_feature_sums(x, row, col, zeros_feat)
  pcnt = _sc_degree_counts(row, zeros_feat, ones_feat)
  return _tc_finish(self_out, psum, pcnt, W_agg.T, b_agg, gamma, beta)


# speedup vs baseline: 9.8693x; 1.0095x over previous
import functools

import jax
import jax.numpy as jnp
from jax import lax
from jax.experimental import pallas as pl
from jax.experimental.pallas import tpu as pltpu
from jax.experimental.pallas import tpu_sc as plsc

N = 10000
E = 320000
D = 128

NC = 2
NS = 16
NW = NC * NS

EDGES_PER_WORKER = E // NW
CHUNK = 80
NCHUNK = EDGES_PER_WORKER // CHUNK
assert NCHUNK * CHUNK == EDGES_PER_WORKER

NPAD = 10240
ROWS_PER_SUBCORE = NPAD // NS
ZCH = ROWS_PER_SUBCORE // CHUNK

_MESH = plsc.VectorSubcoreMesh(
    core_axis_name="c", subcore_axis_name="s", num_cores=NC,
    num_subcores=NS)


def _sc_feature_sums(x, row, col, zeros_feat):
  @functools.partial(
      pl.kernel,
      out_type=jax.ShapeDtypeStruct((NC, NPAD, D), jnp.float32),
      mesh=_MESH,
      scratch_types=[
          [pltpu.VMEM((1, CHUNK), jnp.int32) for _ in range(4)],
          [pltpu.VMEM((1, CHUNK), jnp.int32) for _ in range(4)],
          [pltpu.VMEM((CHUNK, D), jnp.float32) for _ in range(3)],
          pltpu.VMEM_SHARED((NPAD, D), jnp.float32),
          [pltpu.SemaphoreType.DMA for _ in range(4)],
          [pltpu.SemaphoreType.DMA for _ in range(3)],
          [pltpu.SemaphoreType.DMA for _ in range(3)],
      ],
  )
  def sc_kernel(x_hbm, row_hbm, col_hbm, zf_hbm, out_sum,
                colv, rowv, gv, acc, sem_i, sem_g, sem_s):
    cid = lax.axis_index("c")
    sid = lax.axis_index("s")
    wid = cid * NS + sid
    ebase = wid * EDGES_PER_WORKER
    zbase = sid * ROWS_PER_SUBCORE

    pltpu.sync_copy(zf_hbm, gv[0])

    @pl.loop(0, ZCH)
    def _(j):
      pltpu.async_copy(gv[0], acc.at[pl.ds(zbase + j * CHUNK, CHUNK)],
                       sem_g[0])

    @pl.loop(0, ZCH)
    def _(j):
      pltpu.make_async_copy(gv[0], acc.at[pl.ds(zbase, CHUNK)],
                            sem_g[0]).wait()

    plsc.subcore_barrier()

    def issue_idx(c, s):
      off = ebase + c * CHUNK
      pltpu.async_copy(col_hbm.at[pl.ds(off, CHUNK)], colv[s].at[0],
                       sem_i[s])
      pltpu.async_copy(row_hbm.at[pl.ds(off, CHUNK)], rowv[s].at[0],
                       sem_i[s])

    def wait_idx(s):
      pltpu.make_async_copy(col_hbm.at[pl.ds(0, CHUNK)], colv[s].at[0],
                            sem_i[s]).wait()
      pltpu.make_async_copy(row_hbm.at[pl.ds(0, CHUNK)], rowv[s].at[0],
                            sem_i[s]).wait()

    def issue_gather(s, g):
      pltpu.async_copy(x_hbm.at[colv[s].at[0]], gv[g], sem_g[g])

    def wait_gather(s, g):
      pltpu.make_async_copy(x_hbm.at[colv[s].at[0]], gv[g],
                            sem_g[g]).wait()

    def issue_scatter(g, s):
      pltpu.async_copy(gv[g], acc.at[rowv[s].at[0]], sem_s[g], add=True)

    def wait_scatter(g, s):
      pltpu.make_async_copy(gv[g], acc.at[rowv[s].at[0]],
                            sem_s[g]).wait()

    def visit(c, cm, first=False, do_gather=True, do_idx=True):
      i_cur, i_nxt, i_pre = cm % 4, (cm + 1) % 4, (cm + 3) % 4
      g_cur, g_nxt, g_prv = cm % 3, (cm + 1) % 3, (cm + 2) % 3
      if do_gather:
        wait_idx(i_nxt)
        issue_gather(i_nxt, g_nxt)
      wait_gather(i_cur, g_cur)
      issue_scatter(g_cur, i_cur)
      if not first:
        wait_scatter(g_prv, i_pre)
      if do_idx:
        issue_idx(c + 3, i_pre)

    issue_idx(0, 0)
    issue_idx(1, 1)
    issue_idx(2, 2)
    wait_idx(0)
    issue_gather(0, 0)
    visit(0, 0, first=True)

    @pl.loop(0, 10)
    def _(j):
      c = 1 + 12 * j
      for u in range(12):
        visit(c + u, 1 + u)

    visit(121, 121)
    visit(122, 122, do_idx=False)
    visit(123, 123, do_idx=False)
    visit(124, 124, do_gather=False, do_idx=False)
    wait_scatter(124 % 3, 124 % 4)

    plsc.subcore_barrier()

    for j in range(ZCH):
      g = j % 3
      b = zbase + j * CHUNK
      if j >= 3:
        pltpu.make_async_copy(gv[g], out_sum.at[cid, pl.ds(zbase, CHUNK)],
                              sem_s[g]).wait()
      pltpu.sync_copy(acc.at[pl.ds(b, CHUNK)], gv[g])
      pltpu.async_copy(gv[g], out_sum.at[cid, pl.ds(b, CHUNK)], sem_s[g])
    for g in range(3):
      pltpu.make_async_copy(gv[g], out_sum.at[cid, pl.ds(zbase, CHUNK)],
                            sem_s[g]).wait()

  return sc_kernel(x, row, col, zeros_feat)


def _sc_degree_counts(row, zeros_feat, ones_feat):
  @functools.partial(
      pl.kernel,
      out_type=jax.ShapeDtypeStruct((NC, NPAD, D), jnp.float32),
      mesh=_MESH,
      scratch_types=[
          [pltpu.VMEM((1, CHUNK), jnp.int32) for _ in range(4)],
          pltpu.VMEM((CHUNK, D), jnp.float32),
          [pltpu.VMEM((CHUNK, D), jnp.float32) for _ in range(2)],
          pltpu.VMEM_SHARED((NPAD, D), jnp.float32),
          [pltpu.SemaphoreType.DMA for _ in range(4)],
          [pltpu.SemaphoreType.DMA for _ in range(3)],
      ],
  )
  def sc_kernel(row_hbm, zf_hbm, on_hbm, out_cnt,
                rowv, onesv, stg, cnt, sem_i, sem_s):
    cid = lax.axis_index("c")
    sid = lax.axis_index("s")
    wid = cid * NS + sid
    ebase = wid * EDGES_PER_WORKER
    zbase = sid * ROWS_PER_SUBCORE

    pltpu.sync_copy(zf_hbm, onesv)

    @pl.loop(0, ZCH)
    def _(j):
      pltpu.async_copy(onesv, cnt.at[pl.ds(zbase + j * CHUNK, CHUNK)],
                       sem_s[0])

    @pl.loop(0, ZCH)
    def _(j):
      pltpu.make_async_copy(onesv, cnt.at[pl.ds(zbase, CHUNK)],
                            sem_s[0]).wait()

    pltpu.sync_copy(on_hbm, onesv)
    plsc.subcore_barrier()

    def issue_idx(c, s):
      off = ebase + c * CHUNK
      pltpu.async_copy(row_hbm.at[pl.ds(off, CHUNK)], rowv[s].at[0],
                       sem_i[s])

    def wait_idx(s):
      pltpu.make_async_copy(row_hbm.at[pl.ds(0, CHUNK)], rowv[s].at[0],
                            sem_i[s]).wait()

    def issue_scatter(s, g):
      pltpu.async_copy(onesv, cnt.at[rowv[s].at[0]], sem_s[g], add=True)

    def wait_scatter(s, g):
      pltpu.make_async_copy(onesv, cnt.at[rowv[s].at[0]],
                            sem_s[g]).wait()

    def visit(c, cm, first=False, do_idx=True):
      i_cur, i_pre = cm % 4, (cm + 3) % 4
      g_cur, g_prv = cm % 3, (cm + 2) % 3
      wait_idx(i_cur)
      issue_scatter(i_cur, g_cur)
      if not first:
        wait_scatter(i_pre, g_prv)
      if do_idx:
        issue_idx(c + 3, i_pre)

    issue_idx(0, 0)
    issue_idx(1, 1)
    issue_idx(2, 2)
    visit(0, 0, first=True)

    @pl.loop(0, 10)
    def _(j):
      c = 1 + 12 * j
      for u in range(12):
        visit(c + u, 1 + u)

    visit(121, 121)
    visit(122, 122, do_idx=False)
    visit(123, 123, do_idx=False)
    visit(124, 124, do_idx=False)
    wait_scatter(124 % 4, 124 % 3)

    plsc.subcore_barrier()

    bufs = [onesv, stg[0], stg[1]]
    for j in range(ZCH):
      g = j % 3
      b = zbase + j * CHUNK
      if j >= 3:
        pltpu.make_async_copy(bufs[g], out_cnt.at[cid, pl.ds(zbase, CHUNK)],
                              sem_s[g]).wait()
      pltpu.sync_copy(cnt.at[pl.ds(b, CHUNK)], bufs[g])
      pltpu.async_copy(bufs[g], out_cnt.at[cid, pl.ds(b, CHUNK)], sem_s[g])
    for g in range(3):
      pltpu.make_async_copy(bufs[g], out_cnt.at[cid, pl.ds(zbase, CHUNK)],
                            sem_s[g]).wait()

  return sc_kernel(row, zeros_feat, ones_feat)


BLK = 1000


def _tc_self_body(x_ref, ws_ref, bs_ref, out_ref):
  out_ref[...] = (
      jnp.dot(x_ref[...], ws_ref[...], preferred_element_type=jnp.float32)
      + bs_ref[...])


def _tc_self(x, w_self_t, b_self):
  grid = (N // BLK,)
  return pl.pallas_call(
      _tc_self_body,
      grid=grid,
      in_specs=[
          pl.BlockSpec((BLK, D), lambda i: (i, 0)),
          pl.BlockSpec((D, D), lambda i: (0, 0)),
          pl.BlockSpec((1, D), lambda i: (0, 0)),
      ],
      out_specs=pl.BlockSpec((BLK, D), lambda i: (i, 0)),
      out_shape=jax.ShapeDtypeStruct((N, D), jnp.float32),
  )(x, w_self_t, b_self.reshape(1, D))


def _tc_body(sf_ref, ps_ref, pc_ref, wa_ref, ba_ref, g_ref, b_ref, out_ref):
  s = ps_ref[0] + ps_ref[1]
  c = pc_ref[0, :, 0:1] + pc_ref[1, :, 0:1]
  mean = s / (c + 1e-8)
  h = sf_ref[...]
  h = h + jnp.dot(mean, wa_ref[...], preferred_element_type=jnp.float32)
  h = h + ba_ref[...]
  h = jnp.maximum(h, 0.0)
  mu = jnp.mean(h, axis=1, keepdims=True)
  var = jnp.mean((h - mu) ** 2, axis=1, keepdims=True)
  out_ref[...] = (h - mu) * lax.rsqrt(var + 1e-5) * g_ref[...] + b_ref[...]


def _tc_finish(self_out, psum, pcnt, w_agg_t, b_agg, gamma, beta):
  grid = (N // BLK,)
  full128 = pl.BlockSpec((1, D), lambda i: (0, 0))
  return pl.pallas_call(
      _tc_body,
      grid=grid,
      in_specs=[
          pl.BlockSpec((BLK, D), lambda i: (i, 0)),
          pl.BlockSpec((NC, BLK, D), lambda i: (0, i, 0)),
          pl.BlockSpec((NC, BLK, D), lambda i: (0, i, 0)),
          pl.BlockSpec((D, D), lambda i: (0, 0)),
          full128, full128, full128,
      ],
      out_specs=pl.BlockSpec((BLK, D), lambda i: (i, 0)),
      out_shape=jax.ShapeDtypeStruct((N, D), jnp.float32),
  )(self_out, psum, pcnt, w_agg_t,
    b_agg.reshape(1, D), gamma.reshape(1, D), beta.reshape(1, D))


@jax.jit
def kernel(x, edge_index, W_self, b_self, W_agg, b_agg, gamma, beta):
  row = edge_index[0]
  col = edge_index[1]
  zeros_feat = jnp.zeros((CHUNK, D), jnp.float32)
  ones_feat = jnp.ones((CHUNK, D), jnp.float32)
  self_out = _tc_self(x, W_self.T, b_self)
  psum = _sc_feature_sums(x, row, col, zeros_feat)
  pcnt = _sc_degree_counts(row, zeros_feat, ones_feat)
  return _tc_finish(self_out, psum, pcnt, W_agg.T, b_agg, gamma, beta)


# speedup vs baseline: 10.0548x; 1.0188x over previous
import functools

import jax
import jax.numpy as jnp
from jax import lax
from jax.experimental import pallas as pl
from jax.experimental.pallas import tpu as pltpu
from jax.experimental.pallas import tpu_sc as plsc

N = 10000
E = 320000
D = 128

NC = 2
NS = 16
NW = NC * NS

EDGES_PER_WORKER = E // NW
CHUNK = 80
NCHUNK = EDGES_PER_WORKER // CHUNK
assert NCHUNK * CHUNK == EDGES_PER_WORKER

NPAD = 10240
ROWS_PER_SUBCORE = NPAD // NS
ZCH = ROWS_PER_SUBCORE // CHUNK

_MESH = plsc.VectorSubcoreMesh(
    core_axis_name="c", subcore_axis_name="s", num_cores=NC,
    num_subcores=NS)


def _sc_aggregate(x, row, col, zeros_feat, ones_feat):
  @functools.partial(
      pl.kernel,
      out_type=[
          jax.ShapeDtypeStruct((NC, NPAD, D), jnp.float32),
          jax.ShapeDtypeStruct((NC, NPAD, D), jnp.float32),
      ],
      mesh=_MESH,
      scratch_types=[
          [pltpu.VMEM((1, CHUNK), jnp.int32) for _ in range(4)],
          [pltpu.VMEM((1, CHUNK), jnp.int32) for _ in range(4)],
          [pltpu.VMEM((CHUNK, D), jnp.float32) for _ in range(3)],
          pltpu.VMEM_SHARED((NPAD, D), jnp.float32),
          [pltpu.SemaphoreType.DMA for _ in range(4)],
          [pltpu.SemaphoreType.DMA for _ in range(3)],
          [pltpu.SemaphoreType.DMA for _ in range(3)],
      ],
  )
  def sc_kernel(x_hbm, row_hbm, col_hbm, zf_hbm, on_hbm,
                out_sum, out_cnt,
                colv, rowv, gv, acc, sem_i, sem_g, sem_s):
    cid = lax.axis_index("c")
    sid = lax.axis_index("s")
    wid = cid * NS + sid
    ebase = wid * EDGES_PER_WORKER
    zbase = sid * ROWS_PER_SUBCORE

    pltpu.sync_copy(zf_hbm, gv[0])

    @pl.loop(0, ZCH)
    def _(j):
      pltpu.async_copy(gv[0], acc.at[pl.ds(zbase + j * CHUNK, CHUNK)],
                       sem_g[0])

    @pl.loop(0, ZCH)
    def _(j):
      pltpu.make_async_copy(gv[0], acc.at[pl.ds(zbase, CHUNK)],
                            sem_g[0]).wait()

    plsc.subcore_barrier()

    def issue_idx(c, s):
      off = ebase + c * CHUNK
      pltpu.async_copy(col_hbm.at[pl.ds(off, CHUNK)], colv[s].at[0],
                       sem_i[s])
      pltpu.async_copy(row_hbm.at[pl.ds(off, CHUNK)], rowv[s].at[0],
                       sem_i[s])

    def wait_idx(s):
      pltpu.make_async_copy(col_hbm.at[pl.ds(0, CHUNK)], colv[s].at[0],
                            sem_i[s]).wait()
      pltpu.make_async_copy(row_hbm.at[pl.ds(0, CHUNK)], rowv[s].at[0],
                            sem_i[s]).wait()

    def issue_gather(s, g):
      pltpu.async_copy(x_hbm.at[colv[s].at[0]], gv[g], sem_g[g])

    def wait_gather(s, g):
      pltpu.make_async_copy(x_hbm.at[colv[s].at[0]], gv[g],
                            sem_g[g]).wait()

    def issue_scatter(g, s):
      pltpu.async_copy(gv[g], acc.at[rowv[s].at[0]], sem_s[g], add=True)

    def wait_scatter(g, s):
      pltpu.make_async_copy(gv[g], acc.at[rowv[s].at[0]],
                            sem_s[g]).wait()

    def visit(c, cm, first=False, do_gather=True, do_idx=True):
      i_cur, i_nxt, i_pre = cm % 4, (cm + 1) % 4, (cm + 3) % 4
      g_cur, g_nxt, g_prv = cm % 3, (cm + 1) % 3, (cm + 2) % 3
      if do_gather:
        wait_idx(i_nxt)
        issue_gather(i_nxt, g_nxt)
      wait_gather(i_cur, g_cur)
      issue_scatter(g_cur, i_cur)
      if not first:
        wait_scatter(g_prv, i_pre)
      if do_idx:
        issue_idx(c + 3, i_pre)

    issue_idx(0, 0)
    issue_idx(1, 1)
    issue_idx(2, 2)
    wait_idx(0)
    issue_gather(0, 0)
    visit(0, 0, first=True)

    @pl.loop(0, 10)
    def _(j):
      c = 1 + 12 * j
      for u in range(12):
        visit(c + u, 1 + u)

    visit(121, 121)
    visit(122, 122, do_idx=False)
    visit(123, 123, do_idx=False)
    visit(124, 124, do_gather=False, do_idx=False)
    wait_scatter(124 % 3, 124 % 4)

    plsc.subcore_barrier()

    for j in range(ZCH):
      g = j % 3
      b = zbase + j * CHUNK
      if j >= 3:
        pltpu.make_async_copy(gv[g], out_sum.at[cid, pl.ds(zbase, CHUNK)],
                              sem_s[g]).wait()
      pltpu.sync_copy(acc.at[pl.ds(b, CHUNK)], gv[g])
      pltpu.async_copy(gv[g], out_sum.at[cid, pl.ds(b, CHUNK)], sem_s[g])
    for g in range(3):
      pltpu.make_async_copy(gv[g], out_sum.at[cid, pl.ds(zbase, CHUNK)],
                            sem_s[g]).wait()

    pltpu.sync_copy(zf_hbm, gv[0])

    @pl.loop(0, ZCH)
    def _(j):
      pltpu.async_copy(gv[0], acc.at[pl.ds(zbase + j * CHUNK, CHUNK)],
                       sem_g[0])

    @pl.loop(0, ZCH)
    def _(j):
      pltpu.make_async_copy(gv[0], acc.at[pl.ds(zbase, CHUNK)],
                            sem_g[0]).wait()

    pltpu.sync_copy(on_hbm, gv[0])
    plsc.subcore_barrier()

    def cissue_scatter(s, g):
      pltpu.async_copy(gv[0], acc.at[rowv[s].at[0]], sem_s[g], add=True)

    def cwait_scatter(s, g):
      pltpu.make_async_copy(gv[0], acc.at[rowv[s].at[0]],
                            sem_s[g]).wait()

    def cissue_idx(c, s):
      off = ebase + c * CHUNK
      pltpu.async_copy(row_hbm.at[pl.ds(off, CHUNK)], rowv[s].at[0],
                       sem_i[s])

    def cwait_idx(s):
      pltpu.make_async_copy(row_hbm.at[pl.ds(0, CHUNK)], rowv[s].at[0],
                            sem_i[s]).wait()

    def cvisit(c, cm, first=False, do_idx=True):
      i_cur, i_pre = cm % 4, (cm + 3) % 4
      g_cur, g_prv = cm % 3, (cm + 2) % 3
      cwait_idx(i_cur)
      cissue_scatter(i_cur, g_cur)
      if not first:
        cwait_scatter(i_pre, g_prv)
      if do_idx:
        cissue_idx(c + 3, i_pre)

    cissue_idx(0, 0)
    cissue_idx(1, 1)
    cissue_idx(2, 2)
    cvisit(0, 0, first=True)

    @pl.loop(0, 10)
    def _(j):
      c = 1 + 12 * j
      for u in range(12):
        cvisit(c + u, 1 + u)

    cvisit(121, 121)
    cvisit(122, 122, do_idx=False)
    cvisit(123, 123, do_idx=False)
    cvisit(124, 124, do_idx=False)
    cwait_scatter(124 % 4, 124 % 3)

    plsc.subcore_barrier()

    for j in range(ZCH):
      g = 1 + j % 2
      b = zbase + j * CHUNK
      if j >= 2:
        pltpu.make_async_copy(gv[g], out_cnt.at[cid, pl.ds(zbase, CHUNK)],
                              sem_s[g]).wait()
      pltpu.sync_copy(acc.at[pl.ds(b, CHUNK)], gv[g])
      pltpu.async_copy(gv[g], out_cnt.at[cid, pl.ds(b, CHUNK)], sem_s[g])
    for g in (1, 2):
      pltpu.make_async_copy(gv[g], out_cnt.at[cid, pl.ds(zbase, CHUNK)],
                            sem_s[g]).wait()

  return sc_kernel(x, row, col, zeros_feat, ones_feat)


BLK = 1000


def _tc_self_body(x_ref, ws_ref, bs_ref, out_ref):
  out_ref[...] = (
      jnp.dot(x_ref[...], ws_ref[...], preferred_element_type=jnp.float32)
      + bs_ref[...])


def _tc_self(x, w_self_t, b_self):
  grid = (N // BLK,)
  return pl.pallas_call(
      _tc_self_body,
      grid=grid,
      in_specs=[
          pl.BlockSpec((BLK, D), lambda i: (i, 0)),
          pl.BlockSpec((D, D), lambda i: (0, 0)),
          pl.BlockSpec((1, D), lambda i: (0, 0)),
      ],
      out_specs=pl.BlockSpec((BLK, D), lambda i: (i, 0)),
      out_shape=jax.ShapeDtypeStruct((N, D), jnp.float32),
  )(x, w_self_t, b_self.reshape(1, D))


def _tc_body(sf_ref, ps_ref, pc_ref, wa_ref, ba_ref, g_ref, b_ref, out_ref):
  s = ps_ref[0] + ps_ref[1]
  c = pc_ref[0, :, 0:1] + pc_ref[1, :, 0:1]
  mean = s / (c + 1e-8)
  h = sf_ref[...]
  h = h + jnp.dot(mean, wa_ref[...], preferred_element_type=jnp.float32)
  h = h + ba_ref[...]
  h = jnp.maximum(h, 0.0)
  mu = jnp.mean(h, axis=1, keepdims=True)
  var = jnp.mean((h - mu) ** 2, axis=1, keepdims=True)
  out_ref[...] = (h - mu) * lax.rsqrt(var + 1e-5) * g_ref[...] + b_ref[...]


def _tc_finish(self_out, psum, pcnt, w_agg_t, b_agg, gamma, beta):
  grid = (N // BLK,)
  full128 = pl.BlockSpec((1, D), lambda i: (0, 0))
  return pl.pallas_call(
      _tc_body,
      grid=grid,
      in_specs=[
          pl.BlockSpec((BLK, D), lambda i: (i, 0)),
          pl.BlockSpec((NC, BLK, D), lambda i: (0, i, 0)),
          pl.BlockSpec((NC, BLK, D), lambda i: (0, i, 0)),
          pl.BlockSpec((D, D), lambda i: (0, 0)),
          full128, full128, full128,
      ],
      out_specs=pl.BlockSpec((BLK, D), lambda i: (i, 0)),
      out_shape=jax.ShapeDtypeStruct((N, D), jnp.float32),
  )(self_out, psum, pcnt, w_agg_t,
    b_agg.reshape(1, D), gamma.reshape(1, D), beta.reshape(1, D))


@jax.jit
def kernel(x, edge_index, W_self, b_self, W_agg, b_agg, gamma, beta):
  row = edge_index[0]
  col = edge_index[1]
  zeros_feat = jnp.zeros((CHUNK, D), jnp.float32)
  ones_feat = jnp.ones((CHUNK, D), jnp.float32)
  self_out = _tc_self(x, W_self.T, b_self)
  psum, pcnt = _sc_aggregate(x, row, col, zeros_feat, ones_feat)
  return _tc_finish(self_out, psum, pcnt, W_agg.T, b_agg, gamma, beta)
